# baseline (speedup 1.0000x reference)
import functools

import jax
import jax.numpy as jnp
from jax import lax
from jax.experimental import pallas as pl
from jax.experimental.pallas import tpu as pltpu
from jax.experimental.pallas import tpu_sc as plsc

L = 16
PP = 32
SLOTS = 640
GCH = SLOTS // 128


def _bc(s):
    return jnp.broadcast_to(s, (L,))


@functools.lru_cache(maxsize=None)
def _build(B, N, P, J):
    mesh = plsc.VectorSubcoreMesh(core_axis_name="c", subcore_axis_name="s")
    NC = 2
    ROW = P * J * 2
    WIN = 1024
    NCHUNK = 2 * J

    @functools.partial(
        pl.kernel,
        mesh=mesh,
        out_type=jax.ShapeDtypeStruct((2 * B,), jnp.float32),
        compiler_params=pltpu.CompilerParams(needs_layout_passes=False),
        scratch_types=[
            pltpu.VMEM((WIN,), jnp.int32),
            pltpu.VMEM((GCH, 128), jnp.int32),
            pltpu.VMEM((SLOTS,), jnp.float32),
            pltpu.VMEM((NCHUNK * L,), jnp.float32),
            pltpu.VMEM((1, L), jnp.int32),
            pltpu.VMEM((L,), jnp.float32),
            pltpu.SemaphoreType.DMA,
        ],
    )
    def aeloss(tags_hbm, kp_hbm, out_hbm, kp_v, gidx_v, val_v, vis_v,
               oidx_v, oval_v, sem):
        wid = lax.axis_index("s") * NC + lax.axis_index("c")
        zero = jnp.zeros((L,), jnp.float32)
        one = jnp.full((L,), 1.0, jnp.float32)
        lane = lax.iota(jnp.int32, L)

        start = wid * ROW
        shift = start % 8
        astart = pl.multiple_of(start - shift, 8)
        pltpu.sync_copy(kp_hbm.at[pl.ds(astart, WIN)], kp_v)

        base_off = _bc(wid * N).astype(jnp.int32)
        shift_v = _bc(shift).astype(jnp.int32)
        mask14 = lane < (P - L)

        copies = []
        for j in range(J):
            for half in range(2):
                k = 2 * j + half
                if half == 0:
                    pos = shift_v + 34 * lane + 2 * j
                    gi = plsc.load_gather(kp_v, [pos])
                    gf = plsc.load_gather(kp_v, [pos + 1])
                    vis = jnp.where(gf > 0, one, zero)
                else:
                    pos = shift_v + 34 * (L + lane) + 2 * j
                    spos = jnp.where(mask14, pos, 0)
                    gi = plsc.load_gather(kp_v, [spos])
                    gi = jnp.where(mask14, gi, 0)
                    gf = plsc.load_gather(kp_v, [spos + 1])
                    vis = jnp.where(mask14 & (gf > 0), one, zero)
                gidx_v[k // 8, pl.ds((k % 8) * L, L)] = gi + base_off
                vis_v[pl.ds(k * L, L)] = vis
                if k % 8 == 7:
                    r = k // 8
                    copies.append(pltpu.async_copy(
                        tags_hbm.at[gidx_v.at[r]],
                        val_v.at[pl.ds(r * 128, 128)], sem))
        for k in range(NCHUNK, GCH * 8):
            gidx_v[k // 8, pl.ds((k % 8) * L, L)] = base_off
        r = GCH - 1
        copies.append(pltpu.async_copy(
            tags_hbm.at[gidx_v.at[r]], val_v.at[pl.ds(r * 128, 128)], sem))
        for cp in copies:
            cp.wait()

        cnt_lo = cnt_hi = sum_lo = sum_hi = zero
        for j in range(J):
            w_lo = vis_v[pl.ds(j * PP, L)]
            w_hi = vis_v[pl.ds(j * PP + L, L)]
            v_lo = val_v[pl.ds(j * PP, L)]
            v_hi = val_v[pl.ds(j * PP + L, L)]
            cnt_lo = cnt_lo + w_lo
            cnt_hi = cnt_hi + w_hi
            sum_lo = sum_lo + v_lo * w_lo
            sum_hi = sum_hi + v_hi * w_hi
        safe_lo = jnp.maximum(cnt_lo, one)
        safe_hi = jnp.maximum(cnt_hi, one)
        mean_lo = sum_lo / safe_lo
        mean_hi = sum_hi / safe_hi
        valid_lo = jnp.where(cnt_lo > 0, one, zero)
        valid_hi = jnp.where(cnt_hi > 0, one, zero)

        pacc_lo = pacc_hi = zero
        for j in range(J):
            w_lo = vis_v[pl.ds(j * PP, L)]
            w_hi = vis_v[pl.ds(j * PP + L, L)]
            d_lo = val_v[pl.ds(j * PP, L)] - mean_lo
            d_hi = val_v[pl.ds(j * PP + L, L)] - mean_hi
            pacc_lo = pacc_lo + d_lo * d_lo * w_lo
            pacc_hi = pacc_hi + d_hi * d_hi * w_hi
        pull_s = jnp.sum(pacc_lo / safe_lo * valid_lo) + jnp.sum(
            pacc_hi / safe_hi * valid_hi)
        ntags = _bc(jnp.sum(valid_lo) + jnp.sum(valid_hi))

        acc_lo = acc_hi = zero
        for i in range(P):
            sel = lane == (i % L)
            src_m = mean_lo if i < L else mean_hi
            src_v = valid_lo if i < L else valid_hi
            m_i = _bc(jnp.sum(jnp.where(sel, src_m, zero)))
            v_i = _bc(jnp.sum(jnp.where(sel, src_v, zero)))
            d_lo = m_i - mean_lo
            d_hi = m_i - mean_hi
            acc_lo = acc_lo + v_i * jnp.exp(-(d_lo * d_lo)) * valid_lo
            acc_hi = acc_hi + v_i * jnp.exp(-(d_hi * d_hi)) * valid_hi
        push_tot = _bc(jnp.sum(acc_lo) + jnp.sum(acc_hi)) - ntags
        denom = jnp.maximum(ntags * (ntags - one), one)
        push = 0.5 * push_tot / denom
        pull = _bc(pull_s) / jnp.maximum(ntags, one)

        par = lane & 1
        oidx_v[0, :] = _bc(2 * wid) + par
        oval_v[...] = jnp.where(par == 0, pull, push)
        pltpu.async_copy(oval_v, out_hbm.at[oidx_v.at[0]], sem).wait()

    return aeloss


def kernel(input, input1):
    tags = input
    keypoints = input1
    B, N, D = tags.shape
    P, J = keypoints.shape[1], keypoints.shape[2]

    out = _build(B, N, P, J)(
        tags.reshape(B * N),
        keypoints.reshape(B * P * J * 2),
    )
    return out.reshape(B, 2)


# speedup vs baseline: 3.0137x; 3.0137x over previous
import functools

import jax
import jax.numpy as jnp
from jax import lax
from jax.experimental import pallas as pl
from jax.experimental.pallas import tpu as pltpu
from jax.experimental.pallas import tpu_sc as plsc

L = 16
PP = 32
SLOTS = 640
GCH = SLOTS // 128


def _bc(s):
    return jnp.broadcast_to(s, (L,))


@functools.lru_cache(maxsize=None)
def _build(B, N, P, J):
    mesh = plsc.VectorSubcoreMesh(core_axis_name="c", subcore_axis_name="s")
    NC = 2
    ROW = P * J * 2
    WIN = 1024
    NCHUNK = 2 * J

    @functools.partial(
        pl.kernel,
        mesh=mesh,
        out_type=jax.ShapeDtypeStruct((B, L), jnp.float32),
        compiler_params=pltpu.CompilerParams(needs_layout_passes=False),
        scratch_types=[
            pltpu.VMEM((WIN,), jnp.int32),
            pltpu.VMEM((GCH, 128), jnp.int32),
            pltpu.VMEM((SLOTS,), jnp.float32),
            pltpu.VMEM((NCHUNK * L,), jnp.float32),
            pltpu.VMEM((L,), jnp.float32),
            pltpu.SemaphoreType.DMA,
        ],
    )
    def aeloss(tags_hbm, kp_hbm, out_hbm, kp_v, gidx_v, val_v, vis_v,
               oval_v, sem):
        wid = lax.axis_index("s") * NC + lax.axis_index("c")
        zero = jnp.zeros((L,), jnp.float32)
        one = jnp.full((L,), 1.0, jnp.float32)
        lane = lax.iota(jnp.int32, L)

        start = wid * ROW
        shift = start % 8
        astart = pl.multiple_of(start - shift, 8)
        pltpu.sync_copy(kp_hbm.at[pl.ds(astart, WIN)], kp_v)

        base_off = _bc(wid * N).astype(jnp.int32)
        shift_v = _bc(shift).astype(jnp.int32)
        mask14 = lane < (P - L)

        copies = []
        for j in range(J):
            for half in range(2):
                k = 2 * j + half
                if half == 0:
                    pos = shift_v + 34 * lane + 2 * j
                    gi = plsc.load_gather(kp_v, [pos])
                    gf = plsc.load_gather(kp_v, [pos + 1])
                    vis = jnp.where(gf > 0, one, zero)
                else:
                    pos = shift_v + 34 * (L + lane) + 2 * j
                    spos = jnp.where(mask14, pos, 0)
                    gi = plsc.load_gather(kp_v, [spos])
                    gi = jnp.where(mask14, gi, 0)
                    gf = plsc.load_gather(kp_v, [spos + 1])
                    vis = jnp.where(mask14 & (gf > 0), one, zero)
                gidx_v[k // 8, pl.ds((k % 8) * L, L)] = gi + base_off
                vis_v[pl.ds(k * L, L)] = vis
                if k % 8 == 7:
                    r = k // 8
                    copies.append(pltpu.async_copy(
                        tags_hbm.at[gidx_v.at[r]],
                        val_v.at[pl.ds(r * 128, 128)], sem))
        for k in range(NCHUNK, GCH * 8):
            gidx_v[k // 8, pl.ds((k % 8) * L, L)] = base_off
        r = GCH - 1
        copies.append(pltpu.async_copy(
            tags_hbm.at[gidx_v.at[r]], val_v.at[pl.ds(r * 128, 128)], sem))
        for cp in copies:
            cp.wait()

        cnt_lo = cnt_hi = sum_lo = sum_hi = zero
        for j in range(J):
            w_lo = vis_v[pl.ds(j * PP, L)]
            w_hi = vis_v[pl.ds(j * PP + L, L)]
            v_lo = val_v[pl.ds(j * PP, L)]
            v_hi = val_v[pl.ds(j * PP + L, L)]
            cnt_lo = cnt_lo + w_lo
            cnt_hi = cnt_hi + w_hi
            sum_lo = sum_lo + v_lo * w_lo
            sum_hi = sum_hi + v_hi * w_hi
        safe_lo = jnp.maximum(cnt_lo, one)
        safe_hi = jnp.maximum(cnt_hi, one)
        mean_lo = sum_lo / safe_lo
        mean_hi = sum_hi / safe_hi
        valid_lo = jnp.where(cnt_lo > 0, one, zero)
        valid_hi = jnp.where(cnt_hi > 0, one, zero)

        pacc_lo = pacc_hi = zero
        for j in range(J):
            w_lo = vis_v[pl.ds(j * PP, L)]
            w_hi = vis_v[pl.ds(j * PP + L, L)]
            d_lo = val_v[pl.ds(j * PP, L)] - mean_lo
            d_hi = val_v[pl.ds(j * PP + L, L)] - mean_hi
            pacc_lo = pacc_lo + d_lo * d_lo * w_lo
            pacc_hi = pacc_hi + d_hi * d_hi * w_hi
        pull_s = jnp.sum(pacc_lo / safe_lo * valid_lo) + jnp.sum(
            pacc_hi / safe_hi * valid_hi)
        ntags = _bc(jnp.sum(valid_lo) + jnp.sum(valid_hi))

        acc_lo = acc_hi = zero
        for i in range(P):
            sel = lane == (i % L)
            src_m = mean_lo if i < L else mean_hi
            src_v = valid_lo if i < L else valid_hi
            m_i = _bc(jnp.sum(jnp.where(sel, src_m, zero)))
            v_i = _bc(jnp.sum(jnp.where(sel, src_v, zero)))
            d_lo = m_i - mean_lo
            d_hi = m_i - mean_hi
            acc_lo = acc_lo + v_i * jnp.exp(-(d_lo * d_lo)) * valid_lo
            acc_hi = acc_hi + v_i * jnp.exp(-(d_hi * d_hi)) * valid_hi
        push_tot = _bc(jnp.sum(acc_lo) + jnp.sum(acc_hi)) - ntags
        denom = jnp.maximum(ntags * (ntags - one), one)
        push = 0.5 * push_tot / denom
        pull = _bc(pull_s) / jnp.maximum(ntags, one)

        oval_v[...] = jnp.where(lane == 0, pull, jnp.where(lane == 1, push, zero))
        pltpu.sync_copy(oval_v, out_hbm.at[wid])

    return aeloss


def kernel(input, input1):
    tags = input
    keypoints = input1
    B, N, D = tags.shape
    P, J = keypoints.shape[1], keypoints.shape[2]

    out = _build(B, N, P, J)(
        tags.reshape(B * N),
        keypoints.reshape(B * P * J * 2),
    )
    return out[:, :2]
